# baseline (speedup 1.0000x reference)
import jax
import jax.numpy as jnp
from jax import lax
from jax.experimental import pallas as pl
from jax.experimental.pallas import tpu as pltpu

N_NODES = 10000
D_FEAT = 128
N_EDGES = 320000
EB = 2000
NSTEPS = N_EDGES // EB


def _body(src_ref, dst_ref, ef_ref, emb_ref, out_ref):
    @pl.when(pl.program_id(0) == 0)
    def _():
        out_ref[...] = jnp.zeros_like(out_ref)

    def edge(i, carry):
        s = src_ref[0, 0, i]
        d = dst_ref[0, 0, i]
        row = pl.ds(d, 1)
        out_ref[row, :, 0:D_FEAT] = (
            out_ref[row, :, 0:D_FEAT] + emb_ref[pl.ds(s, 1), :, :])
        out_ref[row, :, D_FEAT:D_FEAT + 1] = (
            out_ref[row, :, D_FEAT:D_FEAT + 1] + ef_ref[0, 0, i])
        return carry

    lax.fori_loop(0, EB, edge, 0)


def kernel(edge_index, graph_embedding, e_feat):
    src = edge_index[0].astype(jnp.int32).reshape(NSTEPS, 1, EB)
    dst = edge_index[1].astype(jnp.int32).reshape(NSTEPS, 1, EB)
    ef = e_feat.astype(jnp.float32).reshape(NSTEPS, 1, EB)
    emb = graph_embedding.astype(jnp.float32).reshape(N_NODES, 1, D_FEAT)
    res = pl.pallas_call(
        _body,
        grid=(NSTEPS,),
        in_specs=[
            pl.BlockSpec((1, 1, EB), lambda i: (i, 0, 0),
                         memory_space=pltpu.SMEM),
            pl.BlockSpec((1, 1, EB), lambda i: (i, 0, 0),
                         memory_space=pltpu.SMEM),
            pl.BlockSpec((1, 1, EB), lambda i: (i, 0, 0),
                         memory_space=pltpu.SMEM),
            pl.BlockSpec((N_NODES, 1, D_FEAT), lambda i: (0, 0, 0)),
        ],
        out_specs=pl.BlockSpec((N_NODES, 1, D_FEAT + 1),
                               lambda i: (0, 0, 0)),
        out_shape=jax.ShapeDtypeStruct((N_NODES, 1, D_FEAT + 1),
                                       jnp.float32),
    )(src, dst, ef, emb)
    return res.reshape(N_NODES, D_FEAT + 1)
